# baseline (speedup 1.0000x reference)
import functools

import jax
import jax.numpy as jnp
from jax import lax
from jax.experimental import pallas as pl
from jax.experimental.pallas import tpu as pltpu
from jax.experimental.pallas import tpu_sc as plsc

VOCAB = 1000000
EMB_DIM = 64
OUT_DIM = 1000
BATCH = 4096
SEQ = 200

C0 = 96
C1 = 104

_NC = 2
_NS = 16
_NW = _NC * _NS
_ROWS_PER_W = BATCH // _NW

_VBLK = 1024


def _tr_body(tin_ref, out_ref):
  t = tin_ref[...]
  tt = jnp.transpose(t, (1, 0))
  t3 = tt.reshape(_VBLK // 2, 2, EMB_DIM)
  out_ref[:, 0:EMB_DIM] = t3[:, 0, :]
  out_ref[:, EMB_DIM:2 * EMB_DIM] = t3[:, 1, :]


def _row_major_table(table_t):
  grid = (VOCAB + _VBLK - 1) // _VBLK
  pairs = pl.pallas_call(
      _tr_body,
      grid=(grid,),
      in_specs=[pl.BlockSpec((EMB_DIM, _VBLK), lambda i: (0, i))],
      out_specs=pl.BlockSpec((_VBLK // 2, 2 * EMB_DIM), lambda i: (i, 0)),
      out_shape=jax.ShapeDtypeStruct((VOCAB // 2, 2 * EMB_DIM), jnp.float32),
  )(table_t)
  return pairs.reshape(VOCAB, EMB_DIM)


def _sc_body(table_hbm, idx_hbm, out_hbm, idx_v, gA, gB, obuf, sem):
  wid = lax.axis_index("s") * _NC + lax.axis_index("c")
  base = wid * _ROWS_PER_W

  pltpu.sync_copy(idx_hbm.at[pl.ds(base * SEQ, _ROWS_PER_W * SEQ)], idx_v)

  def gather_pair(b, gbuf):
    c0 = pltpu.make_async_copy(
        table_hbm.at[idx_v.at[pl.ds(b * SEQ, C0)]],
        gbuf.at[pl.ds(0, C0)], sem)
    c1 = pltpu.make_async_copy(
        table_hbm.at[idx_v.at[pl.ds(b * SEQ + C0, C1)]],
        gbuf.at[pl.ds(C0, C1)], sem)
    return c0, c1

  def issue(b, gbuf):
    c0, c1 = gather_pair(b, gbuf)
    c0.start()
    c1.start()

  def wait(b, gbuf):
    c0, c1 = gather_pair(b, gbuf)
    c0.wait()
    c1.wait()

  def sum_row(gbuf, b):
    def body(j, accs):
      a0, a1, a2, a3 = accs
      r0 = j * 8
      for jj in range(8):
        r = r0 + jj
        a0 = a0 + gbuf[r, pl.ds(0, 16)]
        a1 = a1 + gbuf[r, pl.ds(16, 16)]
        a2 = a2 + gbuf[r, pl.ds(32, 16)]
        a3 = a3 + gbuf[r, pl.ds(48, 16)]
      return (a0, a1, a2, a3)

    z = jnp.zeros((16,), jnp.float32)
    a0, a1, a2, a3 = lax.fori_loop(0, SEQ // 8, body, (z, z, z, z))
    obuf[b, pl.ds(0, 16)] = a0
    obuf[b, pl.ds(16, 16)] = a1
    obuf[b, pl.ds(32, 16)] = a2
    obuf[b, pl.ds(48, 16)] = a3

  issue(0, gA)

  def outer(i, carry):
    b0 = 2 * i
    b1 = b0 + 1
    issue(b1, gB)
    wait(b0, gA)
    sum_row(gA, b0)

    @pl.when(i < _ROWS_PER_W // 2 - 1)
    def _():
      issue(b0 + 2, gA)

    wait(b1, gB)
    sum_row(gB, b1)
    return carry

  lax.fori_loop(0, _ROWS_PER_W // 2, outer, 0)

  pltpu.sync_copy(obuf, out_hbm.at[pl.ds(base, _ROWS_PER_W)])


def _sc_gather_sum(table_rm, idx):
  mesh = plsc.VectorSubcoreMesh(core_axis_name="c", subcore_axis_name="s")
  f = functools.partial(
      pl.kernel,
      mesh=mesh,
      compiler_params=pltpu.CompilerParams(use_tc_tiling_on_sc=False),
      out_type=jax.ShapeDtypeStruct((BATCH, EMB_DIM), jnp.float32),
      scratch_types=[
          pltpu.VMEM((_ROWS_PER_W * SEQ,), jnp.int32),
          pltpu.VMEM((SEQ, EMB_DIM), jnp.float32),
          pltpu.VMEM((SEQ, EMB_DIM), jnp.float32),
          pltpu.VMEM((_ROWS_PER_W, EMB_DIM), jnp.float32),
          pltpu.SemaphoreType.DMA,
      ],
  )(_sc_body)
  return f(table_rm, idx)


def _fc_body(emb_ref, w_ref, b_ref, out_ref):
  e = emb_ref[...]
  ss = jnp.sum(e * e, axis=1, keepdims=True)
  scale = 1.0 / jnp.maximum(jnp.sqrt(ss), 1e-12)
  en = e * scale
  acc = lax.dot_general(
      w_ref[...], en, (((1,), (1,)), ((), ())),
      preferred_element_type=jnp.float32,
      precision=lax.Precision.HIGHEST)
  out_ref[...] = acc + b_ref[...]


def _norm_fc(emb, W, b):
  BB = 512
  out_t = pl.pallas_call(
      _fc_body,
      grid=(BATCH // BB,),
      in_specs=[
          pl.BlockSpec((BB, EMB_DIM), lambda i: (i, 0)),
          pl.BlockSpec((OUT_DIM, EMB_DIM), lambda i: (0, 0)),
          pl.BlockSpec((OUT_DIM, 1), lambda i: (0, 0)),
      ],
      out_specs=pl.BlockSpec((OUT_DIM, BB), lambda i: (0, i)),
      out_shape=jax.ShapeDtypeStruct((OUT_DIM, BATCH), jnp.float32),
  )(emb, W, b.reshape(OUT_DIM, 1))
  return out_t.T


def kernel(name_idxs, name_len, desc_idxs, desc_len, union_idxs, union_len,
           table, W, b):
  table_rm = _row_major_table(table.T)
  idx = union_idxs.astype(jnp.int32).reshape(BATCH * SEQ)
  emb_sum = _sc_gather_sum(table_rm, idx)
  return _norm_fc(emb_sum, W, b)


# speedup vs baseline: 1.0982x; 1.0982x over previous
import functools

import jax
import jax.numpy as jnp
from jax import lax
from jax.experimental import pallas as pl
from jax.experimental.pallas import tpu as pltpu
from jax.experimental.pallas import tpu_sc as plsc

VOCAB = 1000000
EMB_DIM = 64
OUT_DIM = 1000
BATCH = 4096
SEQ = 200

C0 = 96
C1 = 104

_NC = 2
_NS = 16
_NW = _NC * _NS
_ROWS_PER_W = BATCH // _NW

_VBLK = 1024
_NBLK = (VOCAB + _VBLK - 1) // _VBLK
VOCAB_F = _NBLK * _VBLK


def _tr_body(tin_ref, out_ref):
  t = tin_ref[...]
  tt = jnp.transpose(t, (1, 0))
  out_ref[...] = jnp.concatenate(
      [tt[0:_VBLK // 2], tt[_VBLK // 2:_VBLK]], axis=1)


def _row_major_table(table_t):
  pairs = pl.pallas_call(
      _tr_body,
      grid=(_NBLK,),
      in_specs=[pl.BlockSpec((EMB_DIM, _VBLK), lambda i: (0, i))],
      out_specs=pl.BlockSpec((_VBLK // 2, 2 * EMB_DIM), lambda i: (i, 0)),
      out_shape=jax.ShapeDtypeStruct((_NBLK * _VBLK // 2, 2 * EMB_DIM),
                                     jnp.float32),
  )(table_t)
  return pairs.reshape(VOCAB_F, EMB_DIM)


def _remap_idx(v):
  return ((v >> 10) << 10) + ((v & 511) << 1) + ((v >> 9) & 1)


def _sc_body(table_hbm, idx_hbm, out_hbm, idx_v, gA, gB, obuf, sem):
  wid = lax.axis_index("s") * _NC + lax.axis_index("c")
  base = wid * _ROWS_PER_W

  pltpu.sync_copy(idx_hbm.at[pl.ds(base * SEQ, _ROWS_PER_W * SEQ)], idx_v)

  def gather_pair(b, gbuf):
    c0 = pltpu.make_async_copy(
        table_hbm.at[idx_v.at[pl.ds(b * SEQ, C0)]],
        gbuf.at[pl.ds(0, C0)], sem)
    c1 = pltpu.make_async_copy(
        table_hbm.at[idx_v.at[pl.ds(b * SEQ + C0, C1)]],
        gbuf.at[pl.ds(C0, C1)], sem)
    return c0, c1

  def issue(b, gbuf):
    c0, c1 = gather_pair(b, gbuf)
    c0.start()
    c1.start()

  def wait(b, gbuf):
    c0, c1 = gather_pair(b, gbuf)
    c0.wait()
    c1.wait()

  def sum_row(gbuf, b):
    def body(j, accs):
      a0, a1, a2, a3 = accs
      r0 = j * 8
      for jj in range(8):
        r = r0 + jj
        a0 = a0 + gbuf[r, pl.ds(0, 16)]
        a1 = a1 + gbuf[r, pl.ds(16, 16)]
        a2 = a2 + gbuf[r, pl.ds(32, 16)]
        a3 = a3 + gbuf[r, pl.ds(48, 16)]
      return (a0, a1, a2, a3)

    z = jnp.zeros((16,), jnp.float32)
    a0, a1, a2, a3 = lax.fori_loop(0, SEQ // 8, body, (z, z, z, z))
    obuf[b, pl.ds(0, 16)] = a0
    obuf[b, pl.ds(16, 16)] = a1
    obuf[b, pl.ds(32, 16)] = a2
    obuf[b, pl.ds(48, 16)] = a3

  issue(0, gA)

  def outer(i, carry):
    b0 = 2 * i
    b1 = b0 + 1
    issue(b1, gB)
    wait(b0, gA)
    sum_row(gA, b0)

    @pl.when(i < _ROWS_PER_W // 2 - 1)
    def _():
      issue(b0 + 2, gA)

    wait(b1, gB)
    sum_row(gB, b1)
    return carry

  lax.fori_loop(0, _ROWS_PER_W // 2, outer, 0)

  pltpu.sync_copy(obuf, out_hbm.at[pl.ds(base, _ROWS_PER_W)])


def _sc_gather_sum(table_rm, idx):
  mesh = plsc.VectorSubcoreMesh(core_axis_name="c", subcore_axis_name="s")
  f = functools.partial(
      pl.kernel,
      mesh=mesh,
      compiler_params=pltpu.CompilerParams(use_tc_tiling_on_sc=False),
      out_type=jax.ShapeDtypeStruct((BATCH, EMB_DIM), jnp.float32),
      scratch_types=[
          pltpu.VMEM((_ROWS_PER_W * SEQ,), jnp.int32),
          pltpu.VMEM((SEQ, EMB_DIM), jnp.float32),
          pltpu.VMEM((SEQ, EMB_DIM), jnp.float32),
          pltpu.VMEM((_ROWS_PER_W, EMB_DIM), jnp.float32),
          pltpu.SemaphoreType.DMA,
      ],
  )(_sc_body)
  return f(table_rm, idx)


def _fc_body(emb_ref, w_ref, b_ref, out_ref):
  e = emb_ref[...]
  ss = jnp.sum(e * e, axis=1, keepdims=True)
  scale = 1.0 / jnp.maximum(jnp.sqrt(ss), 1e-12)
  en = e * scale
  acc = lax.dot_general(
      w_ref[...], en, (((1,), (1,)), ((), ())),
      preferred_element_type=jnp.float32,
      precision=lax.Precision.HIGHEST)
  out_ref[...] = acc + b_ref[...]


def _norm_fc(emb, W, b):
  BB = 512
  out_t = pl.pallas_call(
      _fc_body,
      grid=(BATCH // BB,),
      in_specs=[
          pl.BlockSpec((BB, EMB_DIM), lambda i: (i, 0)),
          pl.BlockSpec((OUT_DIM, EMB_DIM), lambda i: (0, 0)),
          pl.BlockSpec((OUT_DIM, 1), lambda i: (0, 0)),
      ],
      out_specs=pl.BlockSpec((OUT_DIM, BB), lambda i: (0, i)),
      out_shape=jax.ShapeDtypeStruct((OUT_DIM, BATCH), jnp.float32),
  )(emb, W, b.reshape(OUT_DIM, 1))
  return out_t.T


def kernel(name_idxs, name_len, desc_idxs, desc_len, union_idxs, union_len,
           table, W, b):
  table_rm = _row_major_table(table.T)
  idx = _remap_idx(union_idxs.astype(jnp.int32)).reshape(BATCH * SEQ)
  emb_sum = _sc_gather_sum(table_rm, idx)
  return _norm_fc(emb_sum, W, b)


# speedup vs baseline: 2.2435x; 2.0429x over previous
import functools

import jax
import jax.numpy as jnp
from jax import lax
from jax.experimental import pallas as pl
from jax.experimental.pallas import tpu as pltpu
from jax.experimental.pallas import tpu_sc as plsc

VOCAB = 1000000
EMB_DIM = 64
OUT_DIM = 1000
BATCH = 4096
SEQ = 200

C0 = 96
C1 = 104

_NC = 2
_NS = 16
_NW = _NC * _NS
_ROWS_PER_W = BATCH // _NW

_VBLK = 8192
_NBLK = (VOCAB + _VBLK - 1) // _VBLK
VOCAB_F = _NBLK * _VBLK


def _tr_body(tin_ref, out_ref):
  t = tin_ref[...]
  tt = jnp.transpose(t, (1, 0))
  out_ref[...] = jnp.concatenate(
      [tt[0:_VBLK // 2], tt[_VBLK // 2:_VBLK]], axis=1)


def _row_major_table(table_t):
  pairs = pl.pallas_call(
      _tr_body,
      grid=(_NBLK,),
      in_specs=[pl.BlockSpec((EMB_DIM, _VBLK), lambda i: (0, i))],
      out_specs=pl.BlockSpec((_VBLK // 2, 2 * EMB_DIM), lambda i: (i, 0)),
      out_shape=jax.ShapeDtypeStruct((_NBLK * _VBLK // 2, 2 * EMB_DIM),
                                     jnp.float32),
  )(table_t)
  return pairs.reshape(VOCAB_F, EMB_DIM)


def _remap_idx(v):
  return ((v >> 13) << 13) + ((v & 4095) << 1) + ((v >> 12) & 1)


def _sc_body(table_hbm, idx_hbm, out_hbm, idx_v, gA, gB, obuf, sem):
  wid = lax.axis_index("s") * _NC + lax.axis_index("c")
  base = wid * _ROWS_PER_W

  pltpu.sync_copy(idx_hbm.at[pl.ds(base * SEQ, _ROWS_PER_W * SEQ)], idx_v)

  def gather_pair(b, gbuf):
    c0 = pltpu.make_async_copy(
        table_hbm.at[idx_v.at[pl.ds(b * SEQ, C0)]],
        gbuf.at[pl.ds(0, C0)], sem)
    c1 = pltpu.make_async_copy(
        table_hbm.at[idx_v.at[pl.ds(b * SEQ + C0, C1)]],
        gbuf.at[pl.ds(C0, C1)], sem)
    return c0, c1

  def issue(b, gbuf):
    c0, c1 = gather_pair(b, gbuf)
    c0.start()
    c1.start()

  def wait(b, gbuf):
    c0, c1 = gather_pair(b, gbuf)
    c0.wait()
    c1.wait()

  def sum_row(gbuf, b):
    def body(j, accs):
      a0, a1, a2, a3 = accs
      r0 = j * 8
      for jj in range(8):
        r = r0 + jj
        a0 = a0 + gbuf[r, pl.ds(0, 16)]
        a1 = a1 + gbuf[r, pl.ds(16, 16)]
        a2 = a2 + gbuf[r, pl.ds(32, 16)]
        a3 = a3 + gbuf[r, pl.ds(48, 16)]
      return (a0, a1, a2, a3)

    z = jnp.zeros((16,), jnp.float32)
    a0, a1, a2, a3 = lax.fori_loop(0, SEQ // 8, body, (z, z, z, z))
    obuf[b, pl.ds(0, 16)] = a0
    obuf[b, pl.ds(16, 16)] = a1
    obuf[b, pl.ds(32, 16)] = a2
    obuf[b, pl.ds(48, 16)] = a3

  issue(0, gA)

  def outer(i, carry):
    b0 = 2 * i
    b1 = b0 + 1
    issue(b1, gB)
    wait(b0, gA)
    sum_row(gA, b0)

    @pl.when(i < _ROWS_PER_W // 2 - 1)
    def _():
      issue(b0 + 2, gA)

    wait(b1, gB)
    sum_row(gB, b1)
    return carry

  lax.fori_loop(0, _ROWS_PER_W // 2, outer, 0)

  pltpu.sync_copy(obuf, out_hbm.at[pl.ds(base, _ROWS_PER_W)])


def _sc_gather_sum(table_rm, idx):
  mesh = plsc.VectorSubcoreMesh(core_axis_name="c", subcore_axis_name="s")
  f = functools.partial(
      pl.kernel,
      mesh=mesh,
      compiler_params=pltpu.CompilerParams(use_tc_tiling_on_sc=False),
      out_type=jax.ShapeDtypeStruct((BATCH, EMB_DIM), jnp.float32),
      scratch_types=[
          pltpu.VMEM((_ROWS_PER_W * SEQ,), jnp.int32),
          pltpu.VMEM((SEQ, EMB_DIM), jnp.float32),
          pltpu.VMEM((SEQ, EMB_DIM), jnp.float32),
          pltpu.VMEM((_ROWS_PER_W, EMB_DIM), jnp.float32),
          pltpu.SemaphoreType.DMA,
      ],
  )(_sc_body)
  return f(table_rm, idx)


def _fc_body(emb_ref, w_ref, b_ref, out_ref):
  e = emb_ref[...]
  ss = jnp.sum(e * e, axis=1, keepdims=True)
  scale = 1.0 / jnp.maximum(jnp.sqrt(ss), 1e-12)
  en = e * scale
  acc = lax.dot_general(
      w_ref[...], en, (((1,), (1,)), ((), ())),
      preferred_element_type=jnp.float32,
      precision=lax.Precision.HIGHEST)
  out_ref[...] = acc + b_ref[...]


def _norm_fc(emb, W, b):
  BB = 512
  out_t = pl.pallas_call(
      _fc_body,
      grid=(BATCH // BB,),
      in_specs=[
          pl.BlockSpec((BB, EMB_DIM), lambda i: (i, 0)),
          pl.BlockSpec((OUT_DIM, EMB_DIM), lambda i: (0, 0)),
          pl.BlockSpec((OUT_DIM, 1), lambda i: (0, 0)),
      ],
      out_specs=pl.BlockSpec((OUT_DIM, BB), lambda i: (0, i)),
      out_shape=jax.ShapeDtypeStruct((OUT_DIM, BATCH), jnp.float32),
  )(emb, W, b.reshape(OUT_DIM, 1))
  return out_t.T


def kernel(name_idxs, name_len, desc_idxs, desc_len, union_idxs, union_len,
           table, W, b):
  table_rm = _row_major_table(table.T)
  idx = _remap_idx(union_idxs.astype(jnp.int32)).reshape(BATCH * SEQ)
  emb_sum = _sc_gather_sum(table_rm, idx)
  return _norm_fc(emb_sum, W, b)


# speedup vs baseline: 2.4670x; 1.0996x over previous
import functools

import jax
import jax.numpy as jnp
from jax import lax
from jax.experimental import pallas as pl
from jax.experimental.pallas import tpu as pltpu
from jax.experimental.pallas import tpu_sc as plsc

VOCAB = 1000000
EMB_DIM = 64
OUT_DIM = 1000
BATCH = 4096
SEQ = 200

C0 = 96
C1 = 104

_NC = 2
_NS = 16
_NW = _NC * _NS
_ROWS_PER_W = BATCH // _NW

_VBLK = 8192
_NBLK = (VOCAB + _VBLK - 1) // _VBLK
VOCAB_F = _NBLK * _VBLK


def _tr_body(tin_ref, out_ref):
  t = tin_ref[...]
  tt = jnp.transpose(t, (1, 0))
  out_ref[...] = jnp.concatenate(
      [tt[0:_VBLK // 2], tt[_VBLK // 2:_VBLK]], axis=1)


def _row_major_table(table_t):
  pairs = pl.pallas_call(
      _tr_body,
      grid=(_NBLK,),
      in_specs=[pl.BlockSpec((EMB_DIM, _VBLK), lambda i: (0, i))],
      out_specs=pl.BlockSpec((_VBLK // 2, 2 * EMB_DIM), lambda i: (i, 0)),
      out_shape=jax.ShapeDtypeStruct((_NBLK * _VBLK // 2, 2 * EMB_DIM),
                                     jnp.float32),
  )(table_t)
  return pairs.reshape(VOCAB_F, EMB_DIM)


def _remap_idx(v):
  return ((v >> 13) << 13) + ((v & 4095) << 1) + ((v >> 12) & 1)


def _sc_body(table_hbm, idx_hbm, out_hbm, idx_v, g0, g1, g2, g3, obuf, sem):
  wid = lax.axis_index("s") * _NC + lax.axis_index("c")
  base = wid * _ROWS_PER_W

  pltpu.sync_copy(idx_hbm.at[pl.ds(base * SEQ, _ROWS_PER_W * SEQ)], idx_v)

  def gather_pair(b, gbuf):
    c0 = pltpu.make_async_copy(
        table_hbm.at[idx_v.at[pl.ds(b * SEQ, C0)]],
        gbuf.at[pl.ds(0, C0)], sem)
    c1 = pltpu.make_async_copy(
        table_hbm.at[idx_v.at[pl.ds(b * SEQ + C0, C1)]],
        gbuf.at[pl.ds(C0, C1)], sem)
    return c0, c1

  def issue(b, gbuf):
    c0, c1 = gather_pair(b, gbuf)
    c0.start()
    c1.start()

  def wait(b, gbuf):
    c0, c1 = gather_pair(b, gbuf)
    c0.wait()
    c1.wait()

  def sum_row(gbuf, b):
    def body(j, accs):
      a0, a1, a2, a3 = accs
      r0 = j * 8
      for jj in range(8):
        r = r0 + jj
        a0 = a0 + gbuf[r, pl.ds(0, 16)]
        a1 = a1 + gbuf[r, pl.ds(16, 16)]
        a2 = a2 + gbuf[r, pl.ds(32, 16)]
        a3 = a3 + gbuf[r, pl.ds(48, 16)]
      return (a0, a1, a2, a3)

    z = jnp.zeros((16,), jnp.float32)
    a0, a1, a2, a3 = lax.fori_loop(0, SEQ // 8, body, (z, z, z, z))
    obuf[b, pl.ds(0, 16)] = a0
    obuf[b, pl.ds(16, 16)] = a1
    obuf[b, pl.ds(32, 16)] = a2
    obuf[b, pl.ds(48, 16)] = a3

  bufs = (g0, g1, g2, g3)
  issue(0, g0)
  issue(1, g1)
  issue(2, g2)

  def outer(i, carry):
    for ph in range(4):
      b = 4 * i + ph
      nxt = b + 3

      @pl.when(nxt < _ROWS_PER_W)
      def _():
        issue(nxt, bufs[(ph + 3) % 4])

      wait(b, bufs[ph])
      sum_row(bufs[ph], b)
    return carry

  lax.fori_loop(0, _ROWS_PER_W // 4, outer, 0)

  pltpu.sync_copy(obuf, out_hbm.at[pl.ds(base, _ROWS_PER_W)])


def _sc_gather_sum(table_rm, idx):
  mesh = plsc.VectorSubcoreMesh(core_axis_name="c", subcore_axis_name="s")
  f = functools.partial(
      pl.kernel,
      mesh=mesh,
      compiler_params=pltpu.CompilerParams(use_tc_tiling_on_sc=False),
      out_type=jax.ShapeDtypeStruct((BATCH, EMB_DIM), jnp.float32),
      scratch_types=[
          pltpu.VMEM((_ROWS_PER_W * SEQ,), jnp.int32),
          pltpu.VMEM((SEQ, EMB_DIM), jnp.float32),
          pltpu.VMEM((SEQ, EMB_DIM), jnp.float32),
          pltpu.VMEM((SEQ, EMB_DIM), jnp.float32),
          pltpu.VMEM((SEQ, EMB_DIM), jnp.float32),
          pltpu.VMEM((_ROWS_PER_W, EMB_DIM), jnp.float32),
          pltpu.SemaphoreType.DMA,
      ],
  )(_sc_body)
  return f(table_rm, idx)


def _fc_body(emb_ref, w_ref, b_ref, out_ref):
  e = emb_ref[...]
  ss = jnp.sum(e * e, axis=1, keepdims=True)
  scale = 1.0 / jnp.maximum(jnp.sqrt(ss), 1e-12)
  en = e * scale
  acc = lax.dot_general(
      w_ref[...], en, (((1,), (1,)), ((), ())),
      preferred_element_type=jnp.float32,
      precision=lax.Precision.HIGHEST)
  out_ref[...] = acc + b_ref[...]


def _norm_fc(emb, W, b):
  BB = 512
  out_t = pl.pallas_call(
      _fc_body,
      grid=(BATCH // BB,),
      in_specs=[
          pl.BlockSpec((BB, EMB_DIM), lambda i: (i, 0)),
          pl.BlockSpec((OUT_DIM, EMB_DIM), lambda i: (0, 0)),
          pl.BlockSpec((OUT_DIM, 1), lambda i: (0, 0)),
      ],
      out_specs=pl.BlockSpec((OUT_DIM, BB), lambda i: (0, i)),
      out_shape=jax.ShapeDtypeStruct((OUT_DIM, BATCH), jnp.float32),
  )(emb, W, b.reshape(OUT_DIM, 1))
  return out_t.T


def kernel(name_idxs, name_len, desc_idxs, desc_len, union_idxs, union_len,
           table, W, b):
  table_rm = _row_major_table(table.T)
  idx = _remap_idx(union_idxs.astype(jnp.int32)).reshape(BATCH * SEQ)
  emb_sum = _sc_gather_sum(table_rm, idx)
  return _norm_fc(emb_sum, W, b)


# speedup vs baseline: 2.7017x; 1.0951x over previous
import functools

import jax
import jax.numpy as jnp
from jax import lax
from jax.experimental import pallas as pl
from jax.experimental.pallas import tpu as pltpu
from jax.experimental.pallas import tpu_sc as plsc

VOCAB = 1000000
EMB_DIM = 64
OUT_DIM = 1000
BATCH = 4096
SEQ = 200

C0 = 96
C1 = 104

_NC = 2
_NS = 16
_NW = _NC * _NS
_ROWS_PER_W = BATCH // _NW

_VBLK = 16384
_NBLK = (VOCAB + _VBLK - 1) // _VBLK
VOCAB_F = _NBLK * _VBLK


def _tr_body(tin_ref, out_ref):
  t = tin_ref[...]
  tt = jnp.transpose(t, (1, 0))
  out_ref[...] = jnp.concatenate(
      [tt[0:_VBLK // 2], tt[_VBLK // 2:_VBLK]], axis=1)


def _row_major_table(table_t):
  pairs = pl.pallas_call(
      _tr_body,
      grid=(_NBLK,),
      in_specs=[pl.BlockSpec((EMB_DIM, _VBLK), lambda i: (0, i))],
      out_specs=pl.BlockSpec((_VBLK // 2, 2 * EMB_DIM), lambda i: (i, 0)),
      out_shape=jax.ShapeDtypeStruct((_NBLK * _VBLK // 2, 2 * EMB_DIM),
                                     jnp.float32),
  )(table_t)
  return pairs.reshape(VOCAB_F, EMB_DIM)


def _remap_idx(v):
  return ((v >> 14) << 14) + ((v & 8191) << 1) + ((v >> 13) & 1)


def _sc_body(table_hbm, idx_hbm, out_hbm, idx_v, g0, g1, g2, g3, obuf, sem):
  wid = lax.axis_index("s") * _NC + lax.axis_index("c")
  base = wid * _ROWS_PER_W

  pltpu.sync_copy(idx_hbm.at[pl.ds(base * SEQ, _ROWS_PER_W * SEQ)], idx_v)

  def gather_pair(b, gbuf):
    c0 = pltpu.make_async_copy(
        table_hbm.at[idx_v.at[pl.ds(b * SEQ, C0)]],
        gbuf.at[pl.ds(0, C0)], sem)
    c1 = pltpu.make_async_copy(
        table_hbm.at[idx_v.at[pl.ds(b * SEQ + C0, C1)]],
        gbuf.at[pl.ds(C0, C1)], sem)
    return c0, c1

  def issue(b, gbuf):
    c0, c1 = gather_pair(b, gbuf)
    c0.start()
    c1.start()

  def wait(b, gbuf):
    c0, c1 = gather_pair(b, gbuf)
    c0.wait()
    c1.wait()

  def sum_row(gbuf, b):
    def body(j, accs):
      a0, a1, a2, a3 = accs
      r0 = j * 8
      for jj in range(8):
        r = r0 + jj
        a0 = a0 + gbuf[r, pl.ds(0, 16)]
        a1 = a1 + gbuf[r, pl.ds(16, 16)]
        a2 = a2 + gbuf[r, pl.ds(32, 16)]
        a3 = a3 + gbuf[r, pl.ds(48, 16)]
      return (a0, a1, a2, a3)

    z = jnp.zeros((16,), jnp.float32)
    a0, a1, a2, a3 = lax.fori_loop(0, SEQ // 8, body, (z, z, z, z))
    obuf[b, pl.ds(0, 16)] = a0
    obuf[b, pl.ds(16, 16)] = a1
    obuf[b, pl.ds(32, 16)] = a2
    obuf[b, pl.ds(48, 16)] = a3

  bufs = (g0, g1, g2, g3)
  issue(0, g0)
  issue(1, g1)
  issue(2, g2)

  def outer(i, carry):
    for ph in range(4):
      b = 4 * i + ph
      nxt = b + 3

      @pl.when(nxt < _ROWS_PER_W)
      def _():
        issue(nxt, bufs[(ph + 3) % 4])

      wait(b, bufs[ph])
      sum_row(bufs[ph], b)
    return carry

  lax.fori_loop(0, _ROWS_PER_W // 4, outer, 0)

  pltpu.sync_copy(obuf, out_hbm.at[pl.ds(base, _ROWS_PER_W)])


def _sc_gather_sum(table_rm, idx):
  mesh = plsc.VectorSubcoreMesh(core_axis_name="c", subcore_axis_name="s")
  f = functools.partial(
      pl.kernel,
      mesh=mesh,
      compiler_params=pltpu.CompilerParams(use_tc_tiling_on_sc=False),
      out_type=jax.ShapeDtypeStruct((BATCH, EMB_DIM), jnp.float32),
      scratch_types=[
          pltpu.VMEM((_ROWS_PER_W * SEQ,), jnp.int32),
          pltpu.VMEM((SEQ, EMB_DIM), jnp.float32),
          pltpu.VMEM((SEQ, EMB_DIM), jnp.float32),
          pltpu.VMEM((SEQ, EMB_DIM), jnp.float32),
          pltpu.VMEM((SEQ, EMB_DIM), jnp.float32),
          pltpu.VMEM((_ROWS_PER_W, EMB_DIM), jnp.float32),
          pltpu.SemaphoreType.DMA,
      ],
  )(_sc_body)
  return f(table_rm, idx)


def _fc_body(emb_ref, w_ref, b_ref, out_ref):
  e = emb_ref[...]
  ss = jnp.sum(e * e, axis=1, keepdims=True)
  scale = 1.0 / jnp.maximum(jnp.sqrt(ss), 1e-12)
  en = e * scale
  acc = lax.dot_general(
      w_ref[...], en, (((1,), (1,)), ((), ())),
      preferred_element_type=jnp.float32,
      precision=lax.Precision.HIGHEST)
  out_ref[...] = acc + b_ref[...]


def _norm_fc(emb, W, b):
  BB = 512
  out_t = pl.pallas_call(
      _fc_body,
      grid=(BATCH // BB,),
      in_specs=[
          pl.BlockSpec((BB, EMB_DIM), lambda i: (i, 0)),
          pl.BlockSpec((OUT_DIM, EMB_DIM), lambda i: (0, 0)),
          pl.BlockSpec((OUT_DIM, 1), lambda i: (0, 0)),
      ],
      out_specs=pl.BlockSpec((OUT_DIM, BB), lambda i: (0, i)),
      out_shape=jax.ShapeDtypeStruct((OUT_DIM, BATCH), jnp.float32),
  )(emb, W, b.reshape(OUT_DIM, 1))
  return out_t.T


def kernel(name_idxs, name_len, desc_idxs, desc_len, union_idxs, union_len,
           table, W, b):
  table_rm = _row_major_table(table.T)
  idx = _remap_idx(union_idxs.astype(jnp.int32)).reshape(BATCH * SEQ)
  emb_sum = _sc_gather_sum(table_rm, idx)
  return _norm_fc(emb_sum, W, b)


# speedup vs baseline: 2.8213x; 1.0442x over previous
import functools

import jax
import jax.numpy as jnp
from jax import lax
from jax.experimental import pallas as pl
from jax.experimental.pallas import tpu as pltpu
from jax.experimental.pallas import tpu_sc as plsc

VOCAB = 1000000
EMB_DIM = 64
OUT_DIM = 1000
BATCH = 4096
SEQ = 200

C0 = 96
C1 = 104

_NC = 2
_NS = 16
_NW = _NC * _NS
_ROWS_PER_W = BATCH // _NW

_VBLK = 32768
_NBLK = (VOCAB + _VBLK - 1) // _VBLK
VOCAB_F = _NBLK * _VBLK


def _tr_body(tin_ref, out_ref):
  t = tin_ref[...]
  tt = jnp.transpose(t, (1, 0))
  out_ref[...] = jnp.concatenate(
      [tt[0:_VBLK // 2], tt[_VBLK // 2:_VBLK]], axis=1)


def _row_major_table(table_t):
  pairs = pl.pallas_call(
      _tr_body,
      grid=(_NBLK,),
      in_specs=[pl.BlockSpec((EMB_DIM, _VBLK), lambda i: (0, i))],
      out_specs=pl.BlockSpec((_VBLK // 2, 2 * EMB_DIM), lambda i: (i, 0)),
      out_shape=jax.ShapeDtypeStruct((_NBLK * _VBLK // 2, 2 * EMB_DIM),
                                     jnp.float32),
  )(table_t)
  return pairs.reshape(VOCAB_F, EMB_DIM)


def _remap_idx(v):
  return ((v >> 15) << 15) + ((v & 16383) << 1) + ((v >> 14) & 1)


def _sc_body(table_hbm, idx_hbm, out_hbm, idx_v, g0, g1, g2, g3, obuf, sem):
  wid = lax.axis_index("s") * _NC + lax.axis_index("c")
  base = wid * _ROWS_PER_W

  pltpu.sync_copy(idx_hbm.at[pl.ds(base * SEQ, _ROWS_PER_W * SEQ)], idx_v)

  def gather_pair(b, gbuf):
    c0 = pltpu.make_async_copy(
        table_hbm.at[idx_v.at[pl.ds(b * SEQ, C0)]],
        gbuf.at[pl.ds(0, C0)], sem)
    c1 = pltpu.make_async_copy(
        table_hbm.at[idx_v.at[pl.ds(b * SEQ + C0, C1)]],
        gbuf.at[pl.ds(C0, C1)], sem)
    return c0, c1

  def issue(b, gbuf):
    c0, c1 = gather_pair(b, gbuf)
    c0.start()
    c1.start()

  def wait(b, gbuf):
    c0, c1 = gather_pair(b, gbuf)
    c0.wait()
    c1.wait()

  def sum_row(gbuf, b):
    def body(j, accs):
      a0, a1, a2, a3 = accs
      r0 = j * 8
      for jj in range(8):
        r = r0 + jj
        a0 = a0 + gbuf[r, pl.ds(0, 16)]
        a1 = a1 + gbuf[r, pl.ds(16, 16)]
        a2 = a2 + gbuf[r, pl.ds(32, 16)]
        a3 = a3 + gbuf[r, pl.ds(48, 16)]
      return (a0, a1, a2, a3)

    z = jnp.zeros((16,), jnp.float32)
    a0, a1, a2, a3 = lax.fori_loop(0, SEQ // 8, body, (z, z, z, z))
    obuf[b, pl.ds(0, 16)] = a0
    obuf[b, pl.ds(16, 16)] = a1
    obuf[b, pl.ds(32, 16)] = a2
    obuf[b, pl.ds(48, 16)] = a3

  bufs = (g0, g1, g2, g3)
  issue(0, g0)
  issue(1, g1)
  issue(2, g2)

  def outer(i, carry):
    for ph in range(4):
      b = 4 * i + ph
      nxt = b + 3

      @pl.when(nxt < _ROWS_PER_W)
      def _():
        issue(nxt, bufs[(ph + 3) % 4])

      wait(b, bufs[ph])
      sum_row(bufs[ph], b)
    return carry

  lax.fori_loop(0, _ROWS_PER_W // 4, outer, 0)

  pltpu.sync_copy(obuf, out_hbm.at[pl.ds(base, _ROWS_PER_W)])


def _sc_gather_sum(table_rm, idx):
  mesh = plsc.VectorSubcoreMesh(core_axis_name="c", subcore_axis_name="s")
  f = functools.partial(
      pl.kernel,
      mesh=mesh,
      compiler_params=pltpu.CompilerParams(use_tc_tiling_on_sc=False),
      out_type=jax.ShapeDtypeStruct((BATCH, EMB_DIM), jnp.float32),
      scratch_types=[
          pltpu.VMEM((_ROWS_PER_W * SEQ,), jnp.int32),
          pltpu.VMEM((SEQ, EMB_DIM), jnp.float32),
          pltpu.VMEM((SEQ, EMB_DIM), jnp.float32),
          pltpu.VMEM((SEQ, EMB_DIM), jnp.float32),
          pltpu.VMEM((SEQ, EMB_DIM), jnp.float32),
          pltpu.VMEM((_ROWS_PER_W, EMB_DIM), jnp.float32),
          pltpu.SemaphoreType.DMA,
      ],
  )(_sc_body)
  return f(table_rm, idx)


def _fc_body(emb_ref, w_ref, b_ref, out_ref):
  e = emb_ref[...]
  ss = jnp.sum(e * e, axis=1, keepdims=True)
  scale = 1.0 / jnp.maximum(jnp.sqrt(ss), 1e-12)
  en = e * scale
  acc = lax.dot_general(
      w_ref[...], en, (((1,), (1,)), ((), ())),
      preferred_element_type=jnp.float32,
      precision=lax.Precision.HIGHEST)
  out_ref[...] = acc + b_ref[...]


def _norm_fc(emb, W, b):
  BB = 512
  out_t = pl.pallas_call(
      _fc_body,
      grid=(BATCH // BB,),
      in_specs=[
          pl.BlockSpec((BB, EMB_DIM), lambda i: (i, 0)),
          pl.BlockSpec((OUT_DIM, EMB_DIM), lambda i: (0, 0)),
          pl.BlockSpec((OUT_DIM, 1), lambda i: (0, 0)),
      ],
      out_specs=pl.BlockSpec((OUT_DIM, BB), lambda i: (0, i)),
      out_shape=jax.ShapeDtypeStruct((OUT_DIM, BATCH), jnp.float32),
  )(emb, W, b.reshape(OUT_DIM, 1))
  return out_t.T


def kernel(name_idxs, name_len, desc_idxs, desc_len, union_idxs, union_len,
           table, W, b):
  table_rm = _row_major_table(table.T)
  idx = _remap_idx(union_idxs.astype(jnp.int32)).reshape(BATCH * SEQ)
  emb_sum = _sc_gather_sum(table_rm, idx)
  return _norm_fc(emb_sum, W, b)
